# baseline (speedup 1.0000x reference)
import dataclasses

import jax
import jax.numpy as jnp
from jax import lax
from jax.experimental import pallas as pl
from jax.experimental.pallas import tpu as pltpu
from jax.experimental.pallas import tpu_sc as plsc

MU_CONST = 3.5
NUM_LANES = 16
NUM_CORES = 2
NUM_SUBCORES = 16
NUM_WORKERS = NUM_CORES * NUM_SUBCORES
BATCH_SIZE = 16384
FACTORS = 64
WROW = 2 * FACTORS
B_PER_W = BATCH_SIZE // NUM_WORKERS
GATHER_CHUNK = 128
CHUNKS = B_PER_W // GATHER_CHUNK
WIN = 128
WIN_SHIFT = 7
WIN_MASK = WIN - 1


def _sc_kernel(uidx_hbm, iidx_hbm, euw_hbm, eiw_hbm, ubw_hbm, ibw_hbm, out_hbm,
               uidx_v, iidx_v, uwin_v, iwin_v, ubwin_v, ibwin_v,
               upar_v, ipar_v, ulane_v, ilane_v,
               eu0_v, eu1_v, ei0_v, ei1_v, ubw_v, ibw_v, out_v,
               sem_idx, sem_e0, sem_e1, sem_bias):
    wid = lax.axis_index("s") * NUM_CORES + lax.axis_index("c")
    base = wid * B_PER_W

    idx_copies = []
    for j in range(CHUNKS):
        src = pl.ds(base + j * GATHER_CHUNK, GATHER_CHUNK)
        idx_copies.append(pltpu.async_copy(uidx_hbm.at[src], uidx_v.at[j], sem_idx))
        idx_copies.append(pltpu.async_copy(iidx_hbm.at[src], iidx_v.at[j], sem_idx))
    for c in idx_copies:
        c.wait()

    @pl.loop(0, B_PER_W, step=NUM_LANES)
    def _(p):
        j = p // GATHER_CHUNK
        sl = pl.ds(p % GATHER_CHUNK, NUM_LANES)
        fl = pl.ds(p, NUM_LANES)
        u = uidx_v[j, sl]
        i = iidx_v[j, sl]
        uwin_v[j, sl] = u >> 1
        iwin_v[j, sl] = i >> 1
        upar_v[fl] = (u & 1) << 6
        ipar_v[fl] = (i & 1) << 6
        ubwin_v[j, sl] = u >> WIN_SHIFT
        ibwin_v[j, sl] = i >> WIN_SHIFT
        ulane_v[fl] = u & WIN_MASK
        ilane_v[fl] = i & WIN_MASK

    eu_bufs = (eu0_v, eu1_v)
    ei_bufs = (ei0_v, ei1_v)
    sems = (sem_e0, sem_e1)

    def fire(j):
        b = j % 2
        return (pltpu.async_copy(euw_hbm.at[uwin_v.at[j]], eu_bufs[b], sems[b]),
                pltpu.async_copy(eiw_hbm.at[iwin_v.at[j]], ei_bufs[b], sems[b]))

    inflight = [fire(0), fire(1)]

    lane = lax.iota(jnp.int32, NUM_LANES)
    last_lane = lane == (NUM_LANES - 1)

    for j in range(CHUNKS):
        bias_copies = (
            pltpu.async_copy(ubw_hbm.at[ubwin_v.at[j]], ubw_v, sem_bias),
            pltpu.async_copy(ibw_hbm.at[ibwin_v.at[j]], ibw_v, sem_bias),
        )
        for c in inflight[0]:
            c.wait()
        inflight.pop(0)
        eu_v = eu_bufs[j % 2]
        ei_v = ei_bufs[j % 2]

        @pl.loop(0, GATHER_CHUNK)
        def _(q):
            r = j * GATHER_CHUNK + q
            po_u = upar_v[pl.ds(r, NUM_LANES)][0]
            po_i = ipar_v[pl.ds(r, NUM_LANES)][0]
            m = (eu_v[q, pl.ds(po_u, 16)] * ei_v[q, pl.ds(po_i, 16)]
                 + eu_v[q, pl.ds(po_u + 16, 16)] * ei_v[q, pl.ds(po_i + 16, 16)]
                 + eu_v[q, pl.ds(po_u + 32, 16)] * ei_v[q, pl.ds(po_i + 32, 16)]
                 + eu_v[q, pl.ds(po_u + 48, 16)] * ei_v[q, pl.ds(po_i + 48, 16)])
            cc = plsc.cumsum(m)
            plsc.store_compressed(out_v.at[pl.ds(r, NUM_LANES)], cc, mask=last_lane)

        if j + 2 < CHUNKS:
            inflight.append(fire(j + 2))
        for c in bias_copies:
            c.wait()

        @pl.loop(0, GATHER_CHUNK, step=NUM_LANES)
        def _(g):
            rows = lane + g
            fl = pl.ds(j * GATHER_CHUNK + g, NUM_LANES)
            ub = plsc.load_gather(ubw_v, [rows, ulane_v[fl]])
            ib = plsc.load_gather(ibw_v, [rows, ilane_v[fl]])
            out_v[fl] = out_v[fl] + ub + ib + MU_CONST

    pltpu.sync_copy(out_v.at[pl.ds(0, B_PER_W)], out_hbm.at[pl.ds(base, B_PER_W)])


def kernel(user_idx, item_idx, embed_user, embed_item, user_bias, item_bias):
    mesh = plsc.VectorSubcoreMesh(core_axis_name="c", subcore_axis_name="s")
    cp = pltpu.CompilerParams()
    if "needs_layout_passes" in pltpu.CompilerParams.__dataclass_fields__:
        cp = dataclasses.replace(cp, needs_layout_passes=False)
    if "use_tc_tiling_on_sc" in pltpu.CompilerParams.__dataclass_fields__:
        cp = dataclasses.replace(cp, use_tc_tiling_on_sc=True)
    run = pl.kernel(
        _sc_kernel,
        compiler_params=cp,
        out_type=jax.ShapeDtypeStruct((BATCH_SIZE,), jnp.float32),
        mesh=mesh,
        scratch_types=[
            pltpu.VMEM((CHUNKS, GATHER_CHUNK), jnp.int32),
            pltpu.VMEM((CHUNKS, GATHER_CHUNK), jnp.int32),
            pltpu.VMEM((CHUNKS, GATHER_CHUNK), jnp.int32),
            pltpu.VMEM((CHUNKS, GATHER_CHUNK), jnp.int32),
            pltpu.VMEM((CHUNKS, GATHER_CHUNK), jnp.int32),
            pltpu.VMEM((CHUNKS, GATHER_CHUNK), jnp.int32),
            pltpu.VMEM((B_PER_W + NUM_LANES,), jnp.int32),
            pltpu.VMEM((B_PER_W + NUM_LANES,), jnp.int32),
            pltpu.VMEM((B_PER_W,), jnp.int32),
            pltpu.VMEM((B_PER_W,), jnp.int32),
            pltpu.VMEM((GATHER_CHUNK, WROW), jnp.float32),
            pltpu.VMEM((GATHER_CHUNK, WROW), jnp.float32),
            pltpu.VMEM((GATHER_CHUNK, WROW), jnp.float32),
            pltpu.VMEM((GATHER_CHUNK, WROW), jnp.float32),
            pltpu.VMEM((GATHER_CHUNK, WIN), jnp.float32),
            pltpu.VMEM((GATHER_CHUNK, WIN), jnp.float32),
            pltpu.VMEM((B_PER_W + NUM_LANES,), jnp.float32),
            pltpu.SemaphoreType.DMA,
            pltpu.SemaphoreType.DMA,
            pltpu.SemaphoreType.DMA,
            pltpu.SemaphoreType.DMA,
        ],
    )
    ubp = jnp.pad(user_bias[:, 0], (0, (-user_bias.shape[0]) % WIN)).reshape(-1, WIN)
    ibp = jnp.pad(item_bias[:, 0], (0, (-item_bias.shape[0]) % WIN)).reshape(-1, WIN)
    return run(user_idx, item_idx,
               embed_user.reshape(-1, WROW), embed_item.reshape(-1, WROW),
               ubp, ibp)
